# Initial kernel scaffold; baseline (speedup 1.0000x reference)
import jax
import jax.numpy as jnp
from jax.experimental import pallas as pl


def kernel(rs, params, senders_same, receivers_same, senders_anti, receivers_anti, senders_ne, receivers_ne, senders_nn, receivers_nn, senders_en, receivers_en):
    raise NotImplementedError("write your pallas kernel here")


# speedup vs baseline: 38.9859x; 38.9859x over previous
import jax
import jax.numpy as jnp
import numpy as np
from jax import lax
from jax.experimental import pallas as pl
from jax.experimental.pallas import tpu as pltpu
from jax.experimental.pallas import tpu_sc as plsc

N_NUC = 32
N_ELEC = 256
EMB = 128
DFD = 64
CUTOFF = 10.0
EPS = float(np.finfo(np.float32).eps)

NW = 32
E_SAME = 32512
E_ANTI = 32768
E_NE = 8192
EE_PAD = 32768
CH_EE = EE_PAD // NW
CH_NE = E_NE // NW
HEE = N_ELEC * N_ELEC
HNE = N_NUC * N_ELEC
HTOT = HEE + HNE
RB = 32
DFA = 128


def _sc_hist_body(s_same, r_same, s_anti, r_anti, s_ne, r_ne,
                  o_same, o_comb, o_ne, hist, sbuf, rbuf):
    wid = lax.axis_index("s") * 2 + lax.axis_index("c")
    i16 = lax.iota(jnp.int32, 16)
    ones = jnp.ones((16,), jnp.float32)
    z16 = jnp.zeros((16,), jnp.float32)

    def zero_body(i, carry):
        for k in range(8):
            hist[pl.ds(i * 128 + k * 16, 16)] = z16
        return carry
    lax.fori_loop(0, HTOT // 128, zero_body, 0)

    def scatter_edges(sh, rh, chunk, e_real, nsend, off):
        base = wid * chunk
        pltpu.sync_copy(sh.at[pl.ds(base, chunk)], sbuf.at[pl.ds(0, chunk)])
        pltpu.sync_copy(rh.at[pl.ds(base, chunk)], rbuf.at[pl.ds(0, chunk)])

        def body(j, carry):
            vs = sbuf[pl.ds(j * 16, 16)]
            vr = rbuf[pl.ds(j * 16, 16)]
            bins = vr * nsend + vs + off
            mask = (base + j * 16 + i16) < e_real
            plsc.addupdate_scatter(hist, [bins], ones, mask=mask)
            return carry
        lax.fori_loop(0, chunk // 16, body, 0)

    scatter_edges(s_same, r_same, CH_EE, E_SAME, N_ELEC, 0)
    pltpu.sync_copy(hist.at[pl.ds(0, HEE)], o_same.at[wid])
    scatter_edges(s_anti, r_anti, CH_EE, E_ANTI, N_ELEC, 0)
    pltpu.sync_copy(hist.at[pl.ds(0, HEE)], o_comb.at[wid])
    scatter_edges(s_ne, r_ne, CH_NE, E_NE, N_NUC, HEE)
    pltpu.sync_copy(hist.at[pl.ds(HEE, HNE)], o_ne.at[wid])


def _sc_hist(s_same, r_same, s_anti, r_anti, s_ne, r_ne):
    f = pl.kernel(
        _sc_hist_body,
        out_type=(jax.ShapeDtypeStruct((NW, HEE), jnp.float32),
                  jax.ShapeDtypeStruct((NW, HEE), jnp.float32),
                  jax.ShapeDtypeStruct((NW, HNE), jnp.float32)),
        mesh=plsc.VectorSubcoreMesh(core_axis_name="c", subcore_axis_name="s"),
        scratch_types=[pltpu.VMEM((HTOT,), jnp.float32),
                       pltpu.VMEM((CH_EE,), jnp.int32),
                       pltpu.VMEM((CH_EE,), jnp.int32)],
        compiler_params=pltpu.CompilerParams(needs_layout_passes=False),
    )
    return f(s_same, r_same, s_anti, r_anti, s_ne, r_ne)


def _dot(a, b):
    return lax.dot_general(a, b, (((1,), (0,)), ((), ())),
                           precision=lax.Precision.HIGHEST,
                           preferred_element_type=jnp.float32)


def _tc1_body(rs_ref, rsT_ref, coordsT_ref, cs_ref, cc_ref, cn_ref,
              wfsa_ref, wfva_ref, wne_ref, hx_ref, mus_ref, sg2_ref,
              zs_ref, zv_ref):
    r0 = pl.program_id(0) * RB
    cnt_s = jnp.sum(cs_ref[...], axis=0)
    cnt_a = jnp.sum(cc_ref[...], axis=0) - cnt_s
    cnt_n = jnp.sum(cn_ref[...], axis=0)
    mus3 = mus_ref[...].reshape(1, 1, DFA)
    sg23 = sg2_ref[...].reshape(1, 1, DFA)

    def geom(srcT_ref, n):
        dx = [rs_ref[pl.ds(r0, RB), c:c + 1] - srcT_ref[c:c + 1, :]
              for c in range(3)]
        d2 = dx[0] * dx[0] + dx[1] * dx[1] + dx[2] * dx[2]
        d = jnp.sqrt(d2)
        dg = jnp.where(d > EPS, d, EPS)
        dirs = [a / dg for a in dx]
        env = d2 * jnp.exp(-d)
        d3 = d[:, :, None]
        gauss = jnp.exp(-((d3 - mus3) ** 2) / sg23)
        lane = lax.broadcasted_iota(jnp.int32, (RB, n, DFA), 2)
        basis = jnp.where(lane == DFD, 1.0, env[:, :, None] * gauss)
        return basis, dirs

    bee, dir_ee = geom(rsT_ref, N_ELEC)
    for l, w in ((0, cnt_s), (1, cnt_a)):
        zs_ref[l] = _dot(jnp.sum(w[:, :, None] * bee, axis=1), wfsa_ref[l])
        for c in range(3):
            zv_ref[3 * l + c] = _dot(
                jnp.sum((w * dir_ee[c])[:, :, None] * bee, axis=1),
                wfva_ref[l])

    bne, dir_ne = geom(coordsT_ref, N_NUC)
    we = _dot(bne.reshape(RB * N_NUC, DFA), wne_ref[...])
    wh = we.reshape(RB, N_NUC, 2 * EMB) * hx_ref[...][None, :, :]
    zs_ref[2] = jnp.sum(cnt_n[:, :, None] * wh[:, :, :EMB], axis=1)
    for c in range(3):
        zv_ref[6 + c] = jnp.sum(
            (cnt_n * dir_ne[c])[:, :, None] * wh[:, :, EMB:], axis=1)


def _tc2_body(zs_ref, zv_ref, v_ref, u_ref, g1_ref, b1_ref, g2_ref, b2_ref,
              x_ref, outs_ref, outv_ref):
    us = jnp.zeros((N_ELEC, EMB), jnp.float32) + x_ref[...]
    uv = [jnp.zeros((N_ELEC, EMB), jnp.float32) for _ in range(3)]
    for l in range(3):
        vv = [_dot(zv_ref[3 * l + c], v_ref[l]) for c in range(3)]
        uu = [_dot(zv_ref[3 * l + c], u_ref[l]) for c in range(3)]
        norm = jnp.sqrt(vv[0] ** 2 + vv[1] ** 2 + vv[2] ** 2)
        gin = jnp.concatenate([zs_ref[l], norm], axis=1)
        h1 = _dot(gin, g1_ref[l]) + b1_ref[l]
        h1 = h1 / (1.0 + jnp.exp(-h1))
        g = _dot(h1, g2_ref[l]) + b2_ref[l]
        us = us + g[:, 2 * EMB:] * (uu[0] * vv[0] + uu[1] * vv[1]
                                    + uu[2] * vv[2]) + g[:, :EMB]
        for c in range(3):
            uv[c] = uv[c] + uu[c] * g[:, EMB:2 * EMB]
    outs_ref[...] = us
    for c in range(3):
        outv_ref[c] = uv[c]


def _mlp_fold(layers, x):
    n = len(layers)
    for j, (w, b) in enumerate(layers):
        x = x @ w + b
        if j < n - 1:
            x = x * jax.nn.sigmoid(x)
    return x


def kernel(rs, params, senders_same, receivers_same, senders_anti,
           receivers_anti, senders_ne, receivers_ne, senders_nn,
           receivers_nn, senders_en, receivers_en):
    f32 = jnp.float32
    lbls = ("same", "anti", "ne")

    h_e = _mlp_fold(params["h"], params["X"][0])
    hx_n = _mlp_fold(params["h"], params["Y"])
    cols = jnp.concatenate([jnp.arange(EMB), jnp.arange(2 * EMB, 3 * EMB)])

    def aug(wmat, bvec):
        z = jnp.zeros((DFA - DFD - 1, wmat.shape[1]), f32)
        return jnp.concatenate([wmat, bvec[None, :], z], axis=0)

    wfsa, wfva = [], []
    for lbl in ("same", "anti"):
        ww, bw = params["w"][lbl][0]
        wf = ww * h_e[None, :]
        bf = bw * h_e
        wfsa.append(aug(wf[:, :EMB], bf[:EMB]))
        wfva.append(aug(wf[:, 2 * EMB:], bf[2 * EMB:]))
    wfsa = jnp.stack(wfsa)
    wfva = jnp.stack(wfva)
    ww, bw = params["w"]["ne"][0]
    wne_aug = aug(ww[:, cols], bw[cols])
    hx_sel = hx_n[:, cols]

    qs = jnp.linspace(1.0 / (2 * DFD), 1 - 1.0 / (2 * DFD), DFD)
    mus = jnp.pad(CUTOFF * qs ** 2, (0, DFA - DFD))[None, :]
    sg2 = jnp.pad(((1 + CUTOFF * qs) / 7) ** 2, (0, DFA - DFD),
                  constant_values=1.0)[None, :]

    v_all = jnp.stack([params["V"][l] for l in lbls])
    u_all = jnp.stack([params["U"][l] for l in lbls])
    g1_all = jnp.stack([params["g"][l][0][0] for l in lbls])
    b1_all = jnp.stack([params["g"][l][0][1] for l in lbls])[:, None, :]
    g2_all = jnp.stack([params["g"][l][1][0] for l in lbls])
    b2_all = jnp.stack([params["g"][l][1][1] for l in lbls])[:, None, :]

    padn = lambda a: jnp.pad(a, (0, EE_PAD - a.shape[0]))
    hs, hc, hn = _sc_hist(padn(senders_same), padn(receivers_same),
                          senders_anti, receivers_anti,
                          senders_ne, receivers_ne)
    hs3 = hs.reshape(NW, N_ELEC, N_ELEC)
    hc3 = hc.reshape(NW, N_ELEC, N_ELEC)
    hn3 = hn.reshape(NW, N_ELEC, N_NUC)

    nb = N_ELEC // RB
    zs_all, zv_all = pl.pallas_call(
        _tc1_body,
        grid=(nb,),
        in_specs=[
            pl.BlockSpec((N_ELEC, 3), lambda i: (0, 0)),
            pl.BlockSpec((3, N_ELEC), lambda i: (0, 0)),
            pl.BlockSpec((3, N_NUC), lambda i: (0, 0)),
            pl.BlockSpec((NW, RB, N_ELEC), lambda i: (0, i, 0)),
            pl.BlockSpec((NW, RB, N_ELEC), lambda i: (0, i, 0)),
            pl.BlockSpec((NW, RB, N_NUC), lambda i: (0, i, 0)),
            pl.BlockSpec((2, DFA, EMB), lambda i: (0, 0, 0)),
            pl.BlockSpec((2, DFA, EMB), lambda i: (0, 0, 0)),
            pl.BlockSpec((DFA, 2 * EMB), lambda i: (0, 0)),
            pl.BlockSpec((N_NUC, 2 * EMB), lambda i: (0, 0)),
            pl.BlockSpec((1, DFA), lambda i: (0, 0)),
            pl.BlockSpec((1, DFA), lambda i: (0, 0)),
        ],
        out_specs=[
            pl.BlockSpec((3, RB, EMB), lambda i: (0, i, 0)),
            pl.BlockSpec((9, RB, EMB), lambda i: (0, i, 0)),
        ],
        out_shape=[jax.ShapeDtypeStruct((3, N_ELEC, EMB), f32),
                   jax.ShapeDtypeStruct((9, N_ELEC, EMB), f32)],
    )(rs, rs.T, params["coords"].T, hs3, hc3, hn3,
      wfsa, wfva, wne_aug, hx_sel, mus, sg2)

    outs, outv = pl.pallas_call(
        _tc2_body,
        out_shape=[jax.ShapeDtypeStruct((N_ELEC, EMB), f32),
                   jax.ShapeDtypeStruct((3, N_ELEC, EMB), f32)],
    )(zs_all, zv_all, v_all, u_all, g1_all, b1_all, g2_all, b2_all,
      params["X"])

    return jnp.concatenate(
        [outs, jnp.transpose(outv, (1, 2, 0)).reshape(N_ELEC, 3 * EMB)],
        axis=1)


# speedup vs baseline: 78.3899x; 2.0107x over previous
import jax
import jax.numpy as jnp
import numpy as np
from jax import lax
from jax.experimental import pallas as pl
from jax.experimental.pallas import tpu as pltpu
from jax.experimental.pallas import tpu_sc as plsc

N_NUC = 32
N_ELEC = 256
EMB = 128
DFD = 64
CUTOFF = 10.0
EPS = float(np.finfo(np.float32).eps)

NW = 32
E_SAME = 32512
E_ANTI = 32768
E_NE = 8192
EE_PAD = 32768
CH_EE = EE_PAD // NW
CH_NE = E_NE // NW
HEE = N_ELEC * N_ELEC
HNE = N_NUC * N_ELEC
HTOT = HEE + HNE
RB = 32
DFA = 128


def _sc_hist_body(s_same, r_same, s_anti, r_anti, s_ne, r_ne,
                  o_same, o_comb, o_ne, hist, sbuf, rbuf):
    wid = lax.axis_index("s") * 2 + lax.axis_index("c")
    i16 = lax.iota(jnp.int32, 16)
    ones = jnp.ones((16,), jnp.float32)
    z16 = jnp.zeros((16,), jnp.float32)

    def zero_body(i, carry):
        for k in range(8):
            hist[pl.ds(i * 128 + k * 16, 16)] = z16
        return carry
    lax.fori_loop(0, HTOT // 128, zero_body, 0)

    def scatter_edges(sh, rh, chunk, e_real, nsend, off):
        base = wid * chunk
        pltpu.sync_copy(sh.at[pl.ds(base, chunk)], sbuf.at[pl.ds(0, chunk)])
        pltpu.sync_copy(rh.at[pl.ds(base, chunk)], rbuf.at[pl.ds(0, chunk)])

        def body(j, carry):
            vs = sbuf[pl.ds(j * 16, 16)]
            vr = rbuf[pl.ds(j * 16, 16)]
            bins = vr * nsend + vs + off
            mask = (base + j * 16 + i16) < e_real
            plsc.addupdate_scatter(hist, [bins], ones, mask=mask)
            return carry
        lax.fori_loop(0, chunk // 16, body, 0)

    scatter_edges(s_same, r_same, CH_EE, E_SAME, N_ELEC, 0)
    pltpu.sync_copy(hist.at[pl.ds(0, HEE)], o_same.at[wid])
    scatter_edges(s_anti, r_anti, CH_EE, E_ANTI, N_ELEC, 0)
    pltpu.sync_copy(hist.at[pl.ds(0, HEE)], o_comb.at[wid])
    scatter_edges(s_ne, r_ne, CH_NE, E_NE, N_NUC, HEE)
    pltpu.sync_copy(hist.at[pl.ds(HEE, HNE)], o_ne.at[wid])


def _sc_hist(s_same, r_same, s_anti, r_anti, s_ne, r_ne):
    f = pl.kernel(
        _sc_hist_body,
        out_type=(jax.ShapeDtypeStruct((NW, HEE), jnp.float32),
                  jax.ShapeDtypeStruct((NW, HEE), jnp.float32),
                  jax.ShapeDtypeStruct((NW, HNE), jnp.float32)),
        mesh=plsc.VectorSubcoreMesh(core_axis_name="c", subcore_axis_name="s"),
        scratch_types=[pltpu.VMEM((HTOT,), jnp.float32),
                       pltpu.VMEM((CH_EE,), jnp.int32),
                       pltpu.VMEM((CH_EE,), jnp.int32)],
        compiler_params=pltpu.CompilerParams(needs_layout_passes=False),
    )
    return f(s_same, r_same, s_anti, r_anti, s_ne, r_ne)


def _dot(a, b):
    return lax.dot_general(a, b, (((1,), (0,)), ((), ())),
                           precision=lax.Precision.HIGHEST,
                           preferred_element_type=jnp.float32)


def _tc1_body(rs_ref, rsT_ref, coordsT_ref, cs_ref, cc_ref, cn_ref,
              wfs_ref, wfv_ref, bfs_ref, bfv_ref, wne_ref, bne_ref,
              hx_ref, mus_ref, nis_ref, zs_ref, zv_ref):
    r0 = pl.program_id(0) * RB
    cnt_s = jnp.sum(cs_ref[...].reshape(NW, RB, N_ELEC), axis=0)
    cnt_a = jnp.sum(cc_ref[...].reshape(NW, RB, N_ELEC), axis=0) - cnt_s
    cnt_n = jnp.sum(cn_ref[...].reshape(NW, RB, N_NUC), axis=0)

    def dist(srcT_ref):
        dx = [rs_ref[pl.ds(r0, RB), c:c + 1] - srcT_ref[c:c + 1, :]
              for c in range(3)]
        d2 = dx[0] * dx[0] + dx[1] * dx[1] + dx[2] * dx[2]
        d = jnp.sqrt(d2)
        rdg = 1.0 / jnp.where(d > EPS, d, EPS)
        dirs = [a * rdg for a in dx]
        env = d2 * jnp.exp(-d)
        return d, env, dirs

    d, env, dir_ee = dist(rsT_ref)
    t = d[:, None, :] - mus_ref[...].reshape(1, DFD, 1)
    bee = env[:, None, :] * jnp.exp(t * t * nis_ref[...].reshape(1, DFD, 1))
    for l, w in ((0, cnt_s), (1, cnt_a)):
        s0 = jnp.sum(w, axis=1, keepdims=True)
        zs_ref[l] = (_dot(jnp.sum(w[:, None, :] * bee, axis=2), wfs_ref[l])
                     + s0 * bfs_ref[l])
        for c in range(3):
            wd = w * dir_ee[c]
            sd = jnp.sum(wd, axis=1, keepdims=True)
            zv_ref[3 * l + c] = (
                _dot(jnp.sum(wd[:, None, :] * bee, axis=2), wfv_ref[l])
                + sd * bfv_ref[l])

    dn, envn, dir_ne = dist(coordsT_ref)
    tn = dn[:, :, None] - mus_ref[...].reshape(1, 1, DFD)
    bne = envn[:, :, None] * jnp.exp(tn * tn * nis_ref[...].reshape(1, 1, DFD))
    we = _dot(bne.reshape(RB * N_NUC, DFD), wne_ref[...]) + bne_ref[...]
    wh = we.reshape(RB, N_NUC, 2 * EMB) * hx_ref[...][None, :, :]
    zs_ref[2] = jnp.sum(cnt_n[:, :, None] * wh[:, :, :EMB], axis=1)
    for c in range(3):
        zv_ref[6 + c] = jnp.sum(
            (cnt_n * dir_ne[c])[:, :, None] * wh[:, :, EMB:], axis=1)


def _tc2_body(zs_ref, zv_ref, v_ref, u_ref, g1_ref, b1_ref, g2_ref, b2_ref,
              x_ref, outs_ref, outv_ref):
    us = jnp.zeros((N_ELEC, EMB), jnp.float32) + x_ref[...]
    uv = [jnp.zeros((N_ELEC, EMB), jnp.float32) for _ in range(3)]
    for l in range(3):
        vv = [_dot(zv_ref[3 * l + c], v_ref[l]) for c in range(3)]
        uu = [_dot(zv_ref[3 * l + c], u_ref[l]) for c in range(3)]
        norm = jnp.sqrt(vv[0] ** 2 + vv[1] ** 2 + vv[2] ** 2)
        gin = jnp.concatenate([zs_ref[l], norm], axis=1)
        h1 = _dot(gin, g1_ref[l]) + b1_ref[l]
        h1 = h1 / (1.0 + jnp.exp(-h1))
        g = _dot(h1, g2_ref[l]) + b2_ref[l]
        us = us + g[:, 2 * EMB:] * (uu[0] * vv[0] + uu[1] * vv[1]
                                    + uu[2] * vv[2]) + g[:, :EMB]
        for c in range(3):
            uv[c] = uv[c] + uu[c] * g[:, EMB:2 * EMB]
    outs_ref[...] = us
    for c in range(3):
        outv_ref[c] = uv[c]


def _mlp_fold(layers, x):
    n = len(layers)
    for j, (w, b) in enumerate(layers):
        x = x @ w + b
        if j < n - 1:
            x = x * jax.nn.sigmoid(x)
    return x


def kernel(rs, params, senders_same, receivers_same, senders_anti,
           receivers_anti, senders_ne, receivers_ne, senders_nn,
           receivers_nn, senders_en, receivers_en):
    f32 = jnp.float32
    lbls = ("same", "anti", "ne")

    h_e = _mlp_fold(params["h"], params["X"][0])
    hx_n = _mlp_fold(params["h"], params["Y"])
    cols = jnp.concatenate([jnp.arange(EMB), jnp.arange(2 * EMB, 3 * EMB)])

    wfs, wfv, bfs, bfv = [], [], [], []
    for lbl in ("same", "anti"):
        ww, bw = params["w"][lbl][0]
        wf = ww * h_e[None, :]
        bf = bw * h_e
        wfs.append(wf[:, :EMB])
        wfv.append(wf[:, 2 * EMB:])
        bfs.append(bf[None, :EMB])
        bfv.append(bf[None, 2 * EMB:])
    wfs = jnp.stack(wfs)
    wfv = jnp.stack(wfv)
    bfs = jnp.stack(bfs)
    bfv = jnp.stack(bfv)
    ww, bw = params["w"]["ne"][0]
    wne = ww[:, cols]
    bne = bw[cols][None, :]
    hx_sel = hx_n[:, cols]

    qs = jnp.linspace(1.0 / (2 * DFD), 1 - 1.0 / (2 * DFD), DFD)
    mus = (CUTOFF * qs ** 2)[None, :]
    nis = (-1.0 / ((1 + CUTOFF * qs) / 7) ** 2)[None, :]

    v_all = jnp.stack([params["V"][l] for l in lbls])
    u_all = jnp.stack([params["U"][l] for l in lbls])
    g1_all = jnp.stack([params["g"][l][0][0] for l in lbls])
    b1_all = jnp.stack([params["g"][l][0][1] for l in lbls])[:, None, :]
    g2_all = jnp.stack([params["g"][l][1][0] for l in lbls])
    b2_all = jnp.stack([params["g"][l][1][1] for l in lbls])[:, None, :]

    padn = lambda a: jnp.pad(a, (0, EE_PAD - a.shape[0]))
    hs, hc, hn = _sc_hist(padn(senders_same), padn(receivers_same),
                          senders_anti, receivers_anti,
                          senders_ne, receivers_ne)

    nb = N_ELEC // RB
    zs_all, zv_all = pl.pallas_call(
        _tc1_body,
        grid=(nb,),
        in_specs=[
            pl.BlockSpec((N_ELEC, 3), lambda i: (0, 0)),
            pl.BlockSpec((3, N_ELEC), lambda i: (0, 0)),
            pl.BlockSpec((3, N_NUC), lambda i: (0, 0)),
            pl.BlockSpec((NW, RB * N_ELEC), lambda i: (0, i)),
            pl.BlockSpec((NW, RB * N_ELEC), lambda i: (0, i)),
            pl.BlockSpec((NW, RB * N_NUC), lambda i: (0, i)),
            pl.BlockSpec((2, DFD, EMB), lambda i: (0, 0, 0)),
            pl.BlockSpec((2, DFD, EMB), lambda i: (0, 0, 0)),
            pl.BlockSpec((2, 1, EMB), lambda i: (0, 0, 0)),
            pl.BlockSpec((2, 1, EMB), lambda i: (0, 0, 0)),
            pl.BlockSpec((DFD, 2 * EMB), lambda i: (0, 0)),
            pl.BlockSpec((1, 2 * EMB), lambda i: (0, 0)),
            pl.BlockSpec((N_NUC, 2 * EMB), lambda i: (0, 0)),
            pl.BlockSpec((1, DFD), lambda i: (0, 0)),
            pl.BlockSpec((1, DFD), lambda i: (0, 0)),
        ],
        out_specs=[
            pl.BlockSpec((3, RB, EMB), lambda i: (0, i, 0)),
            pl.BlockSpec((9, RB, EMB), lambda i: (0, i, 0)),
        ],
        out_shape=[jax.ShapeDtypeStruct((3, N_ELEC, EMB), f32),
                   jax.ShapeDtypeStruct((9, N_ELEC, EMB), f32)],
    )(rs, rs.T, params["coords"].T, hs, hc, hn,
      wfs, wfv, bfs, bfv, wne, bne, hx_sel, mus, nis)

    outs, outv = pl.pallas_call(
        _tc2_body,
        out_shape=[jax.ShapeDtypeStruct((N_ELEC, EMB), f32),
                   jax.ShapeDtypeStruct((3, N_ELEC, EMB), f32)],
    )(zs_all, zv_all, v_all, u_all, g1_all, b1_all, g2_all, b2_all,
      params["X"])

    return jnp.concatenate(
        [outs, jnp.transpose(outv, (1, 2, 0)).reshape(N_ELEC, 3 * EMB)],
        axis=1)


# speedup vs baseline: 79.0327x; 1.0082x over previous
import jax
import jax.numpy as jnp
import numpy as np
from jax import lax
from jax.experimental import pallas as pl
from jax.experimental.pallas import tpu as pltpu
from jax.experimental.pallas import tpu_sc as plsc

N_NUC = 32
N_ELEC = 256
EMB = 128
DFD = 64
CUTOFF = 10.0
EPS = float(np.finfo(np.float32).eps)

NW = 8
E_SAME = 32512
E_ANTI = 32768
E_NE = 8192
CH_SAME = E_SAME // NW
CH_ANTI = E_ANTI // NW
CH_NE = E_NE // NW
HEE = N_ELEC * N_ELEC
HNE = N_NUC * N_ELEC
HTOT = HEE + HNE
RB = 32

_QS = np.linspace(1.0 / (2 * DFD), 1 - 1.0 / (2 * DFD), DFD, dtype=np.float32)
MUS_NP = (CUTOFF * _QS ** 2).reshape(1, DFD).astype(np.float32)
NIS_NP = (-1.0 / ((1 + CUTOFF * _QS) / 7) ** 2).reshape(1, DFD).astype(np.float32)


def _sc_hist_body(s_same, r_same, s_anti, r_anti, s_ne, r_ne,
                  o_same, o_comb, o_ne, hist, sbuf, rbuf):
    wid = lax.axis_index("s") * 2 + lax.axis_index("c")

    @pl.when(wid < NW)
    def _work():
        ones = jnp.ones((16,), jnp.float32)
        z16 = jnp.zeros((16,), jnp.float32)

        def zero_body(i, carry):
            for k in range(8):
                hist[pl.ds(i * 128 + k * 16, 16)] = z16
            return carry
        lax.fori_loop(0, HTOT // 128, zero_body, 0)

        def scatter_edges(sh, rh, chunk, nsend, off):
            base = wid * chunk
            pltpu.sync_copy(sh.at[pl.ds(base, chunk)],
                            sbuf.at[pl.ds(0, chunk)])
            pltpu.sync_copy(rh.at[pl.ds(base, chunk)],
                            rbuf.at[pl.ds(0, chunk)])

            def body(j, carry):
                vs = sbuf[pl.ds(j * 16, 16)]
                vr = rbuf[pl.ds(j * 16, 16)]
                bins = vr * nsend + vs + off
                plsc.addupdate_scatter(hist, [bins], ones)
                return carry
            lax.fori_loop(0, chunk // 16, body, 0)

        scatter_edges(s_same, r_same, CH_SAME, N_ELEC, 0)
        pltpu.sync_copy(hist.at[pl.ds(0, HEE)], o_same.at[wid])
        scatter_edges(s_anti, r_anti, CH_ANTI, N_ELEC, 0)
        pltpu.sync_copy(hist.at[pl.ds(0, HEE)], o_comb.at[wid])
        scatter_edges(s_ne, r_ne, CH_NE, N_NUC, HEE)
        pltpu.sync_copy(hist.at[pl.ds(HEE, HNE)], o_ne.at[wid])


def _sc_hist(s_same, r_same, s_anti, r_anti, s_ne, r_ne):
    f = pl.kernel(
        _sc_hist_body,
        out_type=(jax.ShapeDtypeStruct((NW, HEE), jnp.float32),
                  jax.ShapeDtypeStruct((NW, HEE), jnp.float32),
                  jax.ShapeDtypeStruct((NW, HNE), jnp.float32)),
        mesh=plsc.VectorSubcoreMesh(core_axis_name="c", subcore_axis_name="s"),
        scratch_types=[pltpu.VMEM((HTOT,), jnp.float32),
                       pltpu.VMEM((CH_ANTI,), jnp.int32),
                       pltpu.VMEM((CH_ANTI,), jnp.int32)],
        compiler_params=pltpu.CompilerParams(needs_layout_passes=False),
    )
    return f(s_same, r_same, s_anti, r_anti, s_ne, r_ne)


def _dot(a, b):
    return lax.dot_general(a, b, (((1,), (0,)), ((), ())),
                           precision=lax.Precision.HIGHEST,
                           preferred_element_type=jnp.float32)


def _silu(x):
    return x / (1.0 + jnp.exp(-x))


def _sel(x):
    return jnp.concatenate([x[:, :EMB], x[:, 2 * EMB:]], axis=1)


def _prep_body(x_ref, y_ref, hw1_ref, hb1_ref, hw2_ref, hb2_ref,
               wws_ref, bws_ref, wwa_ref, bwa_ref, wwn_ref, bwn_ref,
               wfs_s, wfv_s, bfs_s, bfv_s, wfs_a, wfv_a, bfs_a, bfv_a,
               wne_o, bne_o, hx_o):
    xy = jnp.concatenate([x_ref[...], y_ref[...]], axis=0)
    h1 = _silu(_dot(xy, hw1_ref[...]) + hb1_ref[...])
    h2 = _dot(h1, hw2_ref[...]) + hb2_ref[...]
    he = h2[:1]
    for ww_ref, bw_ref, wfs_o, wfv_o, bfs_o, bfv_o in (
            (wws_ref, bws_ref, wfs_s, wfv_s, bfs_s, bfv_s),
            (wwa_ref, bwa_ref, wfs_a, wfv_a, bfs_a, bfv_a)):
        wf = ww_ref[...] * he
        bf = bw_ref[...] * he
        wfs_o[...] = wf[:, :EMB]
        wfv_o[...] = wf[:, 2 * EMB:]
        bfs_o[...] = bf[:, :EMB]
        bfv_o[...] = bf[:, 2 * EMB:]
    wne_o[...] = _sel(wwn_ref[...])
    bne_o[...] = _sel(bwn_ref[...])
    hx_o[...] = _sel(h2[1:])


def _tc1_body(rs_ref, rsT_ref, coordsT_ref, cs_ref, cc_ref, cn_ref,
              wfs_s, wfv_s, bfs_s, bfv_s, wfs_a, wfv_a, bfs_a, bfv_a,
              wne_ref, bne_ref, hx_ref, mus_ref, nis_ref, zs_ref, zv_ref):
    r0 = pl.program_id(0) * RB
    cnt_s = jnp.sum(cs_ref[...].reshape(NW, RB, N_ELEC), axis=0)
    cnt_a = jnp.sum(cc_ref[...].reshape(NW, RB, N_ELEC), axis=0) - cnt_s
    cnt_n = jnp.sum(cn_ref[...].reshape(NW, RB, N_NUC), axis=0)
    mus = mus_ref[...]
    nis = nis_ref[...]

    def dist(srcT_ref):
        dx = [rs_ref[pl.ds(r0, RB), c:c + 1] - srcT_ref[c:c + 1, :]
              for c in range(3)]
        d2 = dx[0] * dx[0] + dx[1] * dx[1] + dx[2] * dx[2]
        d = jnp.sqrt(d2)
        rdg = 1.0 / jnp.where(d > EPS, d, EPS)
        dirs = [a * rdg for a in dx]
        env = d2 * jnp.exp(-d)
        return d, env, dirs

    d, env, dir_ee = dist(rsT_ref)
    t = d[:, None, :] - mus.reshape(1, DFD, 1)
    bee = env[:, None, :] * jnp.exp(t * t * nis.reshape(1, DFD, 1))
    for l, w, wfs, wfv, bfs, bfv in (
            (0, cnt_s, wfs_s, wfv_s, bfs_s, bfv_s),
            (1, cnt_a, wfs_a, wfv_a, bfs_a, bfv_a)):
        s0 = jnp.sum(w, axis=1, keepdims=True)
        zs_ref[l] = (_dot(jnp.sum(w[:, None, :] * bee, axis=2), wfs[...])
                     + s0 * bfs[...])
        for c in range(3):
            wd = w * dir_ee[c]
            sd = jnp.sum(wd, axis=1, keepdims=True)
            zv_ref[3 * l + c] = (
                _dot(jnp.sum(wd[:, None, :] * bee, axis=2), wfv[...])
                + sd * bfv[...])

    dn, envn, dir_ne = dist(coordsT_ref)
    tn = dn[:, :, None] - mus.reshape(1, 1, DFD)
    bne = envn[:, :, None] * jnp.exp(tn * tn * nis.reshape(1, 1, DFD))
    we = _dot(bne.reshape(RB * N_NUC, DFD), wne_ref[...]) + bne_ref[...]
    wh = we.reshape(RB, N_NUC, 2 * EMB) * hx_ref[...][None, :, :]
    zs_ref[2] = jnp.sum(cnt_n[:, :, None] * wh[:, :, :EMB], axis=1)
    for c in range(3):
        zv_ref[6 + c] = jnp.sum(
            (cnt_n * dir_ne[c])[:, :, None] * wh[:, :, EMB:], axis=1)


def _tc2_body(zs_ref, zv_ref, v_s, v_a, v_n, u_s, u_a, u_n,
              g1_s, g1_a, g1_n, b1_s, b1_a, b1_n,
              g2_s, g2_a, g2_n, b2_s, b2_a, b2_n,
              x_ref, outs_ref, outv_ref):
    us = jnp.zeros((N_ELEC, EMB), jnp.float32) + x_ref[...]
    uv = [jnp.zeros((N_ELEC, EMB), jnp.float32) for _ in range(3)]
    for l, v_ref, u_ref, g1_ref, b1_ref, g2_ref, b2_ref in (
            (0, v_s, u_s, g1_s, b1_s, g2_s, b2_s),
            (1, v_a, u_a, g1_a, b1_a, g2_a, b2_a),
            (2, v_n, u_n, g1_n, b1_n, g2_n, b2_n)):
        z3 = zv_ref[3 * l:3 * l + 3].reshape(3 * N_ELEC, EMB)
        vv3 = _dot(z3, v_ref[...])
        uu3 = _dot(z3, u_ref[...])
        vv = [vv3[c * N_ELEC:(c + 1) * N_ELEC] for c in range(3)]
        uu = [uu3[c * N_ELEC:(c + 1) * N_ELEC] for c in range(3)]
        norm = jnp.sqrt(vv[0] ** 2 + vv[1] ** 2 + vv[2] ** 2)
        gin = jnp.concatenate([zs_ref[l], norm], axis=1)
        h1 = _silu(_dot(gin, g1_ref[...]) + b1_ref[...])
        g = _dot(h1, g2_ref[...]) + b2_ref[...]
        us = us + g[:, 2 * EMB:] * (uu[0] * vv[0] + uu[1] * vv[1]
                                    + uu[2] * vv[2]) + g[:, :EMB]
        for c in range(3):
            uv[c] = uv[c] + uu[c] * g[:, EMB:2 * EMB]
    outs_ref[...] = us
    for c in range(3):
        outv_ref[c] = uv[c]


def kernel(rs, params, senders_same, receivers_same, senders_anti,
           receivers_anti, senders_ne, receivers_ne, senders_nn,
           receivers_nn, senders_en, receivers_en):
    f32 = jnp.float32
    row = lambda a: a.reshape(1, -1)

    (hw1, hb1), (hw2, hb2) = params["h"]
    prep_outs = pl.pallas_call(
        _prep_body,
        out_shape=[
            jax.ShapeDtypeStruct((DFD, EMB), f32),
            jax.ShapeDtypeStruct((DFD, EMB), f32),
            jax.ShapeDtypeStruct((1, EMB), f32),
            jax.ShapeDtypeStruct((1, EMB), f32),
            jax.ShapeDtypeStruct((DFD, EMB), f32),
            jax.ShapeDtypeStruct((DFD, EMB), f32),
            jax.ShapeDtypeStruct((1, EMB), f32),
            jax.ShapeDtypeStruct((1, EMB), f32),
            jax.ShapeDtypeStruct((DFD, 2 * EMB), f32),
            jax.ShapeDtypeStruct((1, 2 * EMB), f32),
            jax.ShapeDtypeStruct((N_NUC, 2 * EMB), f32),
        ],
    )(params["X"], params["Y"], hw1, row(hb1), hw2, row(hb2),
      params["w"]["same"][0][0], row(params["w"]["same"][0][1]),
      params["w"]["anti"][0][0], row(params["w"]["anti"][0][1]),
      params["w"]["ne"][0][0], row(params["w"]["ne"][0][1]))

    hs, hc, hn = _sc_hist(senders_same, receivers_same,
                          senders_anti, receivers_anti,
                          senders_ne, receivers_ne)

    nb = N_ELEC // RB
    full = lambda shape: pl.BlockSpec(shape, lambda i: tuple(0 for _ in shape))
    zs_all, zv_all = pl.pallas_call(
        _tc1_body,
        grid=(nb,),
        in_specs=[
            full((N_ELEC, 3)),
            full((3, N_ELEC)),
            full((3, N_NUC)),
            pl.BlockSpec((NW, RB * N_ELEC), lambda i: (0, i)),
            pl.BlockSpec((NW, RB * N_ELEC), lambda i: (0, i)),
            pl.BlockSpec((NW, RB * N_NUC), lambda i: (0, i)),
            full((DFD, EMB)), full((DFD, EMB)),
            full((1, EMB)), full((1, EMB)),
            full((DFD, EMB)), full((DFD, EMB)),
            full((1, EMB)), full((1, EMB)),
            full((DFD, 2 * EMB)), full((1, 2 * EMB)),
            full((N_NUC, 2 * EMB)),
            full((1, DFD)), full((1, DFD)),
        ],
        out_specs=[
            pl.BlockSpec((3, RB, EMB), lambda i: (0, i, 0)),
            pl.BlockSpec((9, RB, EMB), lambda i: (0, i, 0)),
        ],
        out_shape=[jax.ShapeDtypeStruct((3, N_ELEC, EMB), f32),
                   jax.ShapeDtypeStruct((9, N_ELEC, EMB), f32)],
    )(rs, rs.T, params["coords"].T, hs, hc, hn,
      prep_outs[0], prep_outs[1], prep_outs[2], prep_outs[3],
      prep_outs[4], prep_outs[5], prep_outs[6], prep_outs[7],
      prep_outs[8], prep_outs[9], prep_outs[10],
      jnp.asarray(MUS_NP), jnp.asarray(NIS_NP))

    g = params["g"]
    outs, outv = pl.pallas_call(
        _tc2_body,
        out_shape=[jax.ShapeDtypeStruct((N_ELEC, EMB), f32),
                   jax.ShapeDtypeStruct((3, N_ELEC, EMB), f32)],
    )(zs_all, zv_all,
      params["V"]["same"], params["V"]["anti"], params["V"]["ne"],
      params["U"]["same"], params["U"]["anti"], params["U"]["ne"],
      g["same"][0][0], g["anti"][0][0], g["ne"][0][0],
      row(g["same"][0][1]), row(g["anti"][0][1]), row(g["ne"][0][1]),
      g["same"][1][0], g["anti"][1][0], g["ne"][1][0],
      row(g["same"][1][1]), row(g["anti"][1][1]), row(g["ne"][1][1]),
      params["X"])

    return jnp.concatenate(
        [outs, jnp.transpose(outv, (1, 2, 0)).reshape(N_ELEC, 3 * EMB)],
        axis=1)


# speedup vs baseline: 83.2430x; 1.0533x over previous
import jax
import jax.numpy as jnp
import numpy as np
from jax import lax
from jax.experimental import pallas as pl
from jax.experimental.pallas import tpu as pltpu
from jax.experimental.pallas import tpu_sc as plsc

N_NUC = 32
N_ELEC = 256
EMB = 128
DFD = 64
CUTOFF = 10.0
EPS = float(np.finfo(np.float32).eps)

NW = 8
E_SAME = 32512
E_ANTI = 32768
E_NE = 8192
CH_SAME = E_SAME // NW
CH_ANTI = E_ANTI // NW
CH_NE = E_NE // NW
HEE = N_ELEC * N_ELEC
HNE = N_NUC * N_ELEC
HTOT = HEE + HNE
RB = 32

_QS = np.linspace(1.0 / (2 * DFD), 1 - 1.0 / (2 * DFD), DFD, dtype=np.float32)
MUS_NP = (CUTOFF * _QS ** 2).reshape(1, DFD).astype(np.float32)
NIS_NP = (-1.0 / ((1 + CUTOFF * _QS) / 7) ** 2).reshape(1, DFD).astype(np.float32)


def _sc_hist_body(s_same, r_same, s_anti, r_anti, s_ne, r_ne,
                  o_same, o_comb, o_ne, hist, sbuf, rbuf):
    wid = lax.axis_index("s") * 2 + lax.axis_index("c")

    @pl.when(wid < NW)
    def _work():
        ones = jnp.ones((16,), jnp.float32)
        z16 = jnp.zeros((16,), jnp.float32)

        def zero_body(i, carry):
            for k in range(8):
                hist[pl.ds(i * 128 + k * 16, 16)] = z16
            return carry
        lax.fori_loop(0, HTOT // 128, zero_body, 0)

        def scatter_edges(sh, rh, chunk, nsend, off):
            base = wid * chunk
            pltpu.sync_copy(sh.at[pl.ds(base, chunk)],
                            sbuf.at[pl.ds(0, chunk)])
            pltpu.sync_copy(rh.at[pl.ds(base, chunk)],
                            rbuf.at[pl.ds(0, chunk)])

            def body(j, carry):
                vs = sbuf[pl.ds(j * 16, 16)]
                vr = rbuf[pl.ds(j * 16, 16)]
                bins = vr * nsend + vs + off
                plsc.addupdate_scatter(hist, [bins], ones)
                return carry
            lax.fori_loop(0, chunk // 16, body, 0)

        scatter_edges(s_same, r_same, CH_SAME, N_ELEC, 0)
        pltpu.sync_copy(hist.at[pl.ds(0, HEE)], o_same.at[wid])
        scatter_edges(s_anti, r_anti, CH_ANTI, N_ELEC, 0)
        pltpu.sync_copy(hist.at[pl.ds(0, HEE)], o_comb.at[wid])
        scatter_edges(s_ne, r_ne, CH_NE, N_NUC, HEE)
        pltpu.sync_copy(hist.at[pl.ds(HEE, HNE)], o_ne.at[wid])


def _sc_hist(s_same, r_same, s_anti, r_anti, s_ne, r_ne):
    f = pl.kernel(
        _sc_hist_body,
        out_type=(jax.ShapeDtypeStruct((NW, HEE), jnp.float32),
                  jax.ShapeDtypeStruct((NW, HEE), jnp.float32),
                  jax.ShapeDtypeStruct((NW, HNE), jnp.float32)),
        mesh=plsc.VectorSubcoreMesh(core_axis_name="c", subcore_axis_name="s"),
        scratch_types=[pltpu.VMEM((HTOT,), jnp.float32),
                       pltpu.VMEM((CH_ANTI,), jnp.int32),
                       pltpu.VMEM((CH_ANTI,), jnp.int32)],
        compiler_params=pltpu.CompilerParams(needs_layout_passes=False),
    )
    return f(s_same, r_same, s_anti, r_anti, s_ne, r_ne)


def _dot(a, b):
    return lax.dot_general(a, b, (((1,), (0,)), ((), ())),
                           precision=lax.Precision.HIGHEST,
                           preferred_element_type=jnp.float32)


def _silu(x):
    return x / (1.0 + jnp.exp(-x))


def _dot_def(a, b):
    return lax.dot_general(a, b, (((1,), (0,)), ((), ())),
                           precision=lax.Precision.DEFAULT,
                           preferred_element_type=jnp.float32)


def _sel(x):
    return jnp.concatenate([x[:, :EMB], x[:, 2 * EMB:]], axis=1)


def _prep_body(x_ref, y_ref, hw1_ref, hb1_ref, hw2_ref, hb2_ref,
               wws_ref, bws_ref, wwa_ref, bwa_ref, wwn_ref, bwn_ref,
               wfs_s, wfv_s, bfs_s, bfv_s, wfs_a, wfv_a, bfs_a, bfv_a,
               wne_o, bne_o, hx_o):
    xy = jnp.concatenate([x_ref[...], y_ref[...]], axis=0)
    h1 = _silu(_dot_def(xy, hw1_ref[...]) + hb1_ref[...])
    h2 = _dot_def(h1, hw2_ref[...]) + hb2_ref[...]
    he = h2[:1]
    for ww_ref, bw_ref, wfs_o, wfv_o, bfs_o, bfv_o in (
            (wws_ref, bws_ref, wfs_s, wfv_s, bfs_s, bfv_s),
            (wwa_ref, bwa_ref, wfs_a, wfv_a, bfs_a, bfv_a)):
        wf = ww_ref[...] * he
        bf = bw_ref[...] * he
        wfs_o[...] = wf[:, :EMB]
        wfv_o[...] = wf[:, 2 * EMB:]
        bfs_o[...] = bf[:, :EMB]
        bfv_o[...] = bf[:, 2 * EMB:]
    wne_o[...] = _sel(wwn_ref[...])
    bne_o[...] = _sel(bwn_ref[...])
    hx_o[...] = _sel(h2[1:])


def _tc1_body(rs_ref, rsT_ref, coordsT_ref, cs_ref, cc_ref, cn_ref,
              wfs_s, wfv_s, bfs_s, bfv_s, wfs_a, wfv_a, bfs_a, bfv_a,
              wne_ref, bne_ref, hx_ref, mus_ref, nis_ref, zs_ref, zv_ref):
    r0 = pl.program_id(0) * RB
    cnt_s = jnp.sum(cs_ref[...].reshape(NW, RB, N_ELEC), axis=0)
    cnt_a = jnp.sum(cc_ref[...].reshape(NW, RB, N_ELEC), axis=0) - cnt_s
    cnt_n = jnp.sum(cn_ref[...].reshape(NW, RB, N_NUC), axis=0)
    mus = mus_ref[...]
    nis = nis_ref[...]

    def dist(srcT_ref):
        dx = [rs_ref[pl.ds(r0, RB), c:c + 1] - srcT_ref[c:c + 1, :]
              for c in range(3)]
        d2 = dx[0] * dx[0] + dx[1] * dx[1] + dx[2] * dx[2]
        d = jnp.sqrt(d2)
        rdg = 1.0 / jnp.where(d > EPS, d, EPS)
        dirs = [a * rdg for a in dx]
        env = d2 * jnp.exp(-d)
        return d, env, dirs

    d, env, dir_ee = dist(rsT_ref)
    t = d[:, None, :] - mus.reshape(1, DFD, 1)
    bee = env[:, None, :] * jnp.exp(t * t * nis.reshape(1, DFD, 1))
    for l, w, wfs, wfv, bfs, bfv in (
            (0, cnt_s, wfs_s, wfv_s, bfs_s, bfv_s),
            (1, cnt_a, wfs_a, wfv_a, bfs_a, bfv_a)):
        s0 = jnp.sum(w, axis=1, keepdims=True)
        zs_ref[l] = (_dot(jnp.sum(w[:, None, :] * bee, axis=2), wfs[...])
                     + s0 * bfs[...])
        for c in range(3):
            wd = w * dir_ee[c]
            sd = jnp.sum(wd, axis=1, keepdims=True)
            zv_ref[3 * l + c] = (
                _dot(jnp.sum(wd[:, None, :] * bee, axis=2), wfv[...])
                + sd * bfv[...])

    dn, envn, dir_ne = dist(coordsT_ref)
    tn = dn[:, :, None] - mus.reshape(1, 1, DFD)
    bne = envn[:, :, None] * jnp.exp(tn * tn * nis.reshape(1, 1, DFD))
    we = _dot(bne.reshape(RB * N_NUC, DFD), wne_ref[...]) + bne_ref[...]
    wh = we.reshape(RB, N_NUC, 2 * EMB) * hx_ref[...][None, :, :]
    zs_ref[2] = jnp.sum(cnt_n[:, :, None] * wh[:, :, :EMB], axis=1)
    for c in range(3):
        zv_ref[6 + c] = jnp.sum(
            (cnt_n * dir_ne[c])[:, :, None] * wh[:, :, EMB:], axis=1)


def _tc2_body(zs_ref, zv_ref, v_s, v_a, v_n, u_s, u_a, u_n,
              g1_s, g1_a, g1_n, b1_s, b1_a, b1_n,
              g2_s, g2_a, g2_n, b2_s, b2_a, b2_n,
              x_ref, outs_ref, outv_ref):
    us = jnp.zeros((N_ELEC, EMB), jnp.float32) + x_ref[...]
    uv = [jnp.zeros((N_ELEC, EMB), jnp.float32) for _ in range(3)]
    for l, v_ref, u_ref, g1_ref, b1_ref, g2_ref, b2_ref in (
            (0, v_s, u_s, g1_s, b1_s, g2_s, b2_s),
            (1, v_a, u_a, g1_a, b1_a, g2_a, b2_a),
            (2, v_n, u_n, g1_n, b1_n, g2_n, b2_n)):
        z3 = zv_ref[3 * l:3 * l + 3].reshape(3 * N_ELEC, EMB)
        vv3 = _dot_def(z3, v_ref[...])
        uu3 = _dot_def(z3, u_ref[...])
        vv = [vv3[c * N_ELEC:(c + 1) * N_ELEC] for c in range(3)]
        uu = [uu3[c * N_ELEC:(c + 1) * N_ELEC] for c in range(3)]
        norm = jnp.sqrt(vv[0] ** 2 + vv[1] ** 2 + vv[2] ** 2)
        gin = jnp.concatenate([zs_ref[l], norm], axis=1)
        h1 = _silu(_dot_def(gin, g1_ref[...]) + b1_ref[...])
        g = _dot_def(h1, g2_ref[...]) + b2_ref[...]
        us = us + g[:, 2 * EMB:] * (uu[0] * vv[0] + uu[1] * vv[1]
                                    + uu[2] * vv[2]) + g[:, :EMB]
        for c in range(3):
            uv[c] = uv[c] + uu[c] * g[:, EMB:2 * EMB]
    outs_ref[...] = us
    for c in range(3):
        outv_ref[c] = uv[c]


def kernel(rs, params, senders_same, receivers_same, senders_anti,
           receivers_anti, senders_ne, receivers_ne, senders_nn,
           receivers_nn, senders_en, receivers_en):
    f32 = jnp.float32
    row = lambda a: a.reshape(1, -1)

    (hw1, hb1), (hw2, hb2) = params["h"]
    prep_outs = pl.pallas_call(
        _prep_body,
        out_shape=[
            jax.ShapeDtypeStruct((DFD, EMB), f32),
            jax.ShapeDtypeStruct((DFD, EMB), f32),
            jax.ShapeDtypeStruct((1, EMB), f32),
            jax.ShapeDtypeStruct((1, EMB), f32),
            jax.ShapeDtypeStruct((DFD, EMB), f32),
            jax.ShapeDtypeStruct((DFD, EMB), f32),
            jax.ShapeDtypeStruct((1, EMB), f32),
            jax.ShapeDtypeStruct((1, EMB), f32),
            jax.ShapeDtypeStruct((DFD, 2 * EMB), f32),
            jax.ShapeDtypeStruct((1, 2 * EMB), f32),
            jax.ShapeDtypeStruct((N_NUC, 2 * EMB), f32),
        ],
    )(params["X"], params["Y"], hw1, row(hb1), hw2, row(hb2),
      params["w"]["same"][0][0], row(params["w"]["same"][0][1]),
      params["w"]["anti"][0][0], row(params["w"]["anti"][0][1]),
      params["w"]["ne"][0][0], row(params["w"]["ne"][0][1]))

    hs, hc, hn = _sc_hist(senders_same, receivers_same,
                          senders_anti, receivers_anti,
                          senders_ne, receivers_ne)

    nb = N_ELEC // RB
    full = lambda shape: pl.BlockSpec(shape, lambda i: tuple(0 for _ in shape))
    zs_all, zv_all = pl.pallas_call(
        _tc1_body,
        grid=(nb,),
        in_specs=[
            full((N_ELEC, 3)),
            full((3, N_ELEC)),
            full((3, N_NUC)),
            pl.BlockSpec((NW, RB * N_ELEC), lambda i: (0, i)),
            pl.BlockSpec((NW, RB * N_ELEC), lambda i: (0, i)),
            pl.BlockSpec((NW, RB * N_NUC), lambda i: (0, i)),
            full((DFD, EMB)), full((DFD, EMB)),
            full((1, EMB)), full((1, EMB)),
            full((DFD, EMB)), full((DFD, EMB)),
            full((1, EMB)), full((1, EMB)),
            full((DFD, 2 * EMB)), full((1, 2 * EMB)),
            full((N_NUC, 2 * EMB)),
            full((1, DFD)), full((1, DFD)),
        ],
        out_specs=[
            pl.BlockSpec((3, RB, EMB), lambda i: (0, i, 0)),
            pl.BlockSpec((9, RB, EMB), lambda i: (0, i, 0)),
        ],
        out_shape=[jax.ShapeDtypeStruct((3, N_ELEC, EMB), f32),
                   jax.ShapeDtypeStruct((9, N_ELEC, EMB), f32)],
    )(rs, rs.T, params["coords"].T, hs, hc, hn,
      prep_outs[0], prep_outs[1], prep_outs[2], prep_outs[3],
      prep_outs[4], prep_outs[5], prep_outs[6], prep_outs[7],
      prep_outs[8], prep_outs[9], prep_outs[10],
      jnp.asarray(MUS_NP), jnp.asarray(NIS_NP))

    g = params["g"]
    outs, outv = pl.pallas_call(
        _tc2_body,
        out_shape=[jax.ShapeDtypeStruct((N_ELEC, EMB), f32),
                   jax.ShapeDtypeStruct((3, N_ELEC, EMB), f32)],
    )(zs_all, zv_all,
      params["V"]["same"], params["V"]["anti"], params["V"]["ne"],
      params["U"]["same"], params["U"]["anti"], params["U"]["ne"],
      g["same"][0][0], g["anti"][0][0], g["ne"][0][0],
      row(g["same"][0][1]), row(g["anti"][0][1]), row(g["ne"][0][1]),
      g["same"][1][0], g["anti"][1][0], g["ne"][1][0],
      row(g["same"][1][1]), row(g["anti"][1][1]), row(g["ne"][1][1]),
      params["X"])

    return jnp.concatenate(
        [outs, jnp.transpose(outv, (1, 2, 0)).reshape(N_ELEC, 3 * EMB)],
        axis=1)
